# baseline (speedup 1.0000x reference)
import functools

import jax
import jax.numpy as jnp
from jax import lax
from jax.experimental import pallas as pl
from jax.experimental.pallas import tpu as pltpu
from jax.experimental.pallas import tpu_sc as plsc

_NW = 32
_LANES = 16
_CH = 4


def _sc_gather_sum(ntab, etab, nbrs_flat, deg, n, n_pad, d):
    ipc = _CH * deg
    nchunks = n_pad // _CH
    real_chunks = n // _CH
    per_w = nchunks // _NW
    nouter = per_w // 2

    mesh = plsc.VectorSubcoreMesh(core_axis_name="c", subcore_axis_name="s")

    @functools.partial(
        pl.kernel,
        mesh=mesh,
        compiler_params=pltpu.CompilerParams(needs_layout_passes=False),
        out_type=[jax.ShapeDtypeStruct((n_pad, d), jnp.float32),
                  jax.ShapeDtypeStruct((n_pad, d), jnp.float32)],
        scratch_types=[
            pltpu.VMEM((2 * ipc,), jnp.int32),
            pltpu.VMEM((2 * ipc,), jnp.int32),
            pltpu.VMEM((ipc,), jnp.int32),
            pltpu.VMEM((ipc,), jnp.int32),
            pltpu.VMEM((ipc,), jnp.int32),
            pltpu.VMEM((ipc,), jnp.int32),
            pltpu.VMEM((ipc, d), jnp.float32),
            pltpu.VMEM((ipc, d), jnp.float32),
            pltpu.VMEM((ipc, d), jnp.float32),
            pltpu.VMEM((ipc, d), jnp.float32),
            pltpu.VMEM((_CH, d), jnp.float32),
            pltpu.VMEM((_CH, d), jnp.float32),
            pltpu.SemaphoreType.DMA,
            pltpu.SemaphoreType.DMA,
            pltpu.SemaphoreType.DMA,
            pltpu.SemaphoreType.DMA,
        ],
    )
    def sc_kernel(node_hbm, edge_hbm, nbrs_hbm, gn_hbm, ge_hbm,
                  raw0, raw1, nidx0, nidx1, eidx0, eidx1,
                  nrows0, nrows1, erows0, erows1,
                  outn_v, oute_v, semn0, semn1, seme0, seme1):
        raw = (raw0, raw1)
        nidx = (nidx0, nidx1)
        eidx = (eidx0, eidx1)
        nrows = (nrows0, nrows1)
        erows = (erows0, erows1)
        semn = (semn0, semn1)
        seme = (seme0, seme1)
        wid = lax.axis_index("s") * 2 + lax.axis_index("c")
        start = wid * per_w
        start_row = start * _CH
        iota2 = lax.iota(jnp.int32, _LANES) * 2

        def load_idx(k, buf):
            c = start + k
            pltpu.sync_copy(nbrs_hbm.at[pl.ds(c * 2 * ipc, 2 * ipc)], raw[buf])
            for t in range(ipc // _LANES):
                pair = iota2 + (2 * _LANES * t)
                nidx[buf][pl.ds(_LANES * t, _LANES)] = plsc.load_gather(
                    raw[buf], [pair])
                eidx[buf][pl.ds(_LANES * t, _LANES)] = plsc.load_gather(
                    raw[buf], [pair + 1])

        def fire(buf):
            pltpu.async_copy(node_hbm.at[nidx[buf]], nrows[buf], semn[buf])
            pltpu.async_copy(edge_hbm.at[eidx[buf]], erows[buf], seme[buf])

        def drain(buf):
            pltpu.make_async_copy(node_hbm.at[nidx[buf]], nrows[buf],
                                  semn[buf]).wait()
            pltpu.make_async_copy(edge_hbm.at[eidx[buf]], erows[buf],
                                  seme[buf]).wait()

        def compute(b, k):
            def node_body(node, carry):
                def row_body(r, accs):
                    row = node * deg + r
                    nvec = d // _LANES
                    upd_n = tuple(
                        accs[j] + nrows[b][row, pl.ds(_LANES * j, _LANES)]
                        for j in range(nvec))
                    upd_e = tuple(
                        accs[nvec + j] + erows[b][row, pl.ds(_LANES * j, _LANES)]
                        for j in range(nvec))
                    return upd_n + upd_e
                nvec = d // _LANES
                zeros = tuple(jnp.zeros((_LANES,), jnp.float32)
                              for _ in range(2 * nvec))
                accs = lax.fori_loop(0, deg, row_body, zeros)
                for j in range(nvec):
                    outn_v[node, pl.ds(_LANES * j, _LANES)] = accs[j]
                    oute_v[node, pl.ds(_LANES * j, _LANES)] = accs[nvec + j]
                return carry
            lax.fori_loop(0, _CH, node_body, 0)
            pltpu.sync_copy(outn_v, gn_hbm.at[pl.ds(start_row + k * _CH, _CH), :])
            pltpu.sync_copy(oute_v, ge_hbm.at[pl.ds(start_row + k * _CH, _CH), :])

        def valid(k):
            return start + k < real_chunks

        load_idx(0, 0)
        fire(0)

        def outer(i, carry):
            for b in range(2):
                k = 2 * i + b
                if b == 0:
                    @pl.when(valid(k + 1))
                    def _():
                        load_idx(k + 1, 1)
                        fire(1)
                else:
                    @pl.when((i < nouter - 1) & valid(k + 1))
                    def _():
                        load_idx(k + 1, 0)
                        fire(0)

                @pl.when(valid(k))
                def _():
                    drain(b)
                    compute(b, k)
            return carry

        lax.fori_loop(0, nouter, outer, 0)

    return sc_kernel(ntab, etab, nbrs_flat)


def _tc_finish(node_feats, gn, ge, w, b2, scale):
    n, d = node_feats.shape
    h = w.shape[0]
    br = 1000
    grid = (n // br,)

    def body(x_ref, gn_ref, ge_ref, w_ref, b_ref, o_ref):
        xs = x_ref[...] + gn_ref[...]
        acc = lax.dot_general(xs, w_ref[...], (((1,), (1,)), ((), ())),
                              preferred_element_type=jnp.float32)
        o_ref[...] = acc + ge_ref[...] + scale * b_ref[...]

    return pl.pallas_call(
        body,
        grid=grid,
        in_specs=[
            pl.BlockSpec((br, d), lambda i: (i, 0)),
            pl.BlockSpec((br, h), lambda i: (i, 0)),
            pl.BlockSpec((br, h), lambda i: (i, 0)),
            pl.BlockSpec((h, d), lambda i: (0, 0)),
            pl.BlockSpec((1, h), lambda i: (0, 0)),
        ],
        out_specs=pl.BlockSpec((br, h), lambda i: (i, 0)),
        out_shape=jax.ShapeDtypeStruct((n, h), jnp.float32),
    )(node_feats, gn, ge, w, b2)


def kernel(node_feats, edge_feats, neighbors, W, b):
    n, d = node_feats.shape
    deg = neighbors.shape[1]
    chunks_per_w = -(-(n // _CH) // (2 * _NW)) * 2
    n_pad = chunks_per_w * _NW * _CH
    nbrs_flat = neighbors.reshape(n * deg * 2)
    gn, ge = _sc_gather_sum(node_feats, edge_feats, nbrs_flat,
                            deg, n, n_pad, d)
    return _tc_finish(node_feats, gn, ge, W, b.reshape(1, -1), float(deg + 1))


# speedup vs baseline: 1.5670x; 1.5670x over previous
import functools

import jax
import jax.numpy as jnp
from jax import lax
from jax.experimental import pallas as pl
from jax.experimental.pallas import tpu as pltpu
from jax.experimental.pallas import tpu_sc as plsc

_NW = 32
_LANES = 16
_CH = 4


def _sc_gather_sum(ntab, etab, nidx2d, eidx2d, deg, n_pad, d):
    ipc = _CH * deg
    nchunks = n_pad // _CH
    per_w = nchunks // _NW
    nouter = per_w // 2

    mesh = plsc.VectorSubcoreMesh(core_axis_name="c", subcore_axis_name="s")

    @functools.partial(
        pl.kernel,
        mesh=mesh,
        out_type=[jax.ShapeDtypeStruct((n_pad, d), jnp.float32),
                  jax.ShapeDtypeStruct((n_pad, d), jnp.float32)],
        scratch_types=[
            pltpu.VMEM((ipc,), jnp.int32),
            pltpu.VMEM((ipc,), jnp.int32),
            pltpu.VMEM((ipc,), jnp.int32),
            pltpu.VMEM((ipc,), jnp.int32),
            pltpu.VMEM((ipc, d), jnp.float32),
            pltpu.VMEM((ipc, d), jnp.float32),
            pltpu.VMEM((ipc, d), jnp.float32),
            pltpu.VMEM((ipc, d), jnp.float32),
            pltpu.VMEM((_CH, d), jnp.float32),
            pltpu.VMEM((_CH, d), jnp.float32),
            pltpu.SemaphoreType.DMA,
            pltpu.SemaphoreType.DMA,
            pltpu.SemaphoreType.DMA,
            pltpu.SemaphoreType.DMA,
        ],
    )
    def sc_kernel(node_hbm, edge_hbm, nidx_hbm, eidx_hbm, gn_hbm, ge_hbm,
                  nidx0, nidx1, eidx0, eidx1, nrows0, nrows1, erows0, erows1,
                  outn_v, oute_v, semn0, semn1, seme0, seme1):
        nidx = (nidx0, nidx1)
        eidx = (eidx0, eidx1)
        nrows = (nrows0, nrows1)
        erows = (erows0, erows1)
        semn = (semn0, semn1)
        seme = (seme0, seme1)
        wid = lax.axis_index("s") * 2 + lax.axis_index("c")
        start = wid * per_w
        start_row = start * _CH

        def load_idx(k, buf):
            pltpu.sync_copy(nidx_hbm.at[start + k], nidx[buf])
            pltpu.sync_copy(eidx_hbm.at[start + k], eidx[buf])

        def fire(buf):
            pltpu.async_copy(node_hbm.at[nidx[buf]], nrows[buf], semn[buf])
            pltpu.async_copy(edge_hbm.at[eidx[buf]], erows[buf], seme[buf])

        def drain(buf):
            pltpu.make_async_copy(node_hbm.at[nidx[buf]], nrows[buf],
                                  semn[buf]).wait()
            pltpu.make_async_copy(edge_hbm.at[eidx[buf]], erows[buf],
                                  seme[buf]).wait()

        def compute(b, k):
            def node_body(node, carry):
                nvec = d // _LANES
                def row_body(r, accs):
                    row = node * deg + r
                    upd_n = tuple(
                        accs[j] + nrows[b][row, pl.ds(_LANES * j, _LANES)]
                        for j in range(nvec))
                    upd_e = tuple(
                        accs[nvec + j] + erows[b][row, pl.ds(_LANES * j, _LANES)]
                        for j in range(nvec))
                    return upd_n + upd_e
                zeros = tuple(jnp.zeros((_LANES,), jnp.float32)
                              for _ in range(2 * nvec))
                accs = lax.fori_loop(0, deg, row_body, zeros)
                for j in range(nvec):
                    outn_v[node, pl.ds(_LANES * j, _LANES)] = accs[j]
                    oute_v[node, pl.ds(_LANES * j, _LANES)] = accs[nvec + j]
                return carry
            lax.fori_loop(0, _CH, node_body, 0)
            pltpu.sync_copy(outn_v, gn_hbm.at[pl.ds(start_row + k * _CH, _CH), :])
            pltpu.sync_copy(oute_v, ge_hbm.at[pl.ds(start_row + k * _CH, _CH), :])

        load_idx(0, 0)
        fire(0)

        def outer(i, carry):
            for b in range(2):
                k = 2 * i + b
                if b == 0:
                    load_idx(k + 1, 1)
                    fire(1)
                else:
                    @pl.when(i < nouter - 1)
                    def _():
                        load_idx(k + 1, 0)
                        fire(0)
                drain(b)
                compute(b, k)
            return carry

        lax.fori_loop(0, nouter, outer, 0)

    return sc_kernel(ntab, etab, nidx2d, eidx2d)


def _tc_finish(node_feats, gn, ge, w, b2, scale):
    n, d = node_feats.shape
    h = w.shape[0]
    br = 1000
    grid = (n // br,)

    def body(x_ref, gn_ref, ge_ref, w_ref, b_ref, o_ref):
        xs = x_ref[...] + gn_ref[...]
        acc = lax.dot_general(xs, w_ref[...], (((1,), (1,)), ((), ())),
                              preferred_element_type=jnp.float32)
        o_ref[...] = acc + ge_ref[...] + scale * b_ref[...]

    return pl.pallas_call(
        body,
        grid=grid,
        in_specs=[
            pl.BlockSpec((br, d), lambda i: (i, 0)),
            pl.BlockSpec((br, h), lambda i: (i, 0)),
            pl.BlockSpec((br, h), lambda i: (i, 0)),
            pl.BlockSpec((h, d), lambda i: (0, 0)),
            pl.BlockSpec((1, h), lambda i: (0, 0)),
        ],
        out_specs=pl.BlockSpec((br, h), lambda i: (i, 0)),
        out_shape=jax.ShapeDtypeStruct((n, h), jnp.float32),
    )(node_feats, gn, ge, w, b2)


def kernel(node_feats, edge_feats, neighbors, W, b):
    n, d = node_feats.shape
    deg = neighbors.shape[1]
    ipc = _CH * deg
    chunks_per_w = -(-(n // _CH) // (2 * _NW)) * 2
    n_pad = chunks_per_w * _NW * _CH
    nchunks = n_pad // _CH
    nidx = neighbors[:, :, 0].reshape(n * deg)
    eidx = neighbors[:, :, 1].reshape(n * deg)
    pad = nchunks * ipc - n * deg
    tail = jnp.arange(pad, dtype=jnp.int32) % n
    nidx2d = jnp.concatenate([nidx, tail]).reshape(nchunks, ipc)
    eidx2d = jnp.concatenate([eidx, tail]).reshape(nchunks, ipc)
    gn, ge = _sc_gather_sum(node_feats, edge_feats, nidx2d, eidx2d,
                            deg, n_pad, d)
    return _tc_finish(node_feats, gn, ge, W, b.reshape(1, -1), float(deg + 1))


# speedup vs baseline: 1.5828x; 1.0100x over previous
import functools

import jax
import jax.numpy as jnp
from jax import lax
from jax.experimental import pallas as pl
from jax.experimental.pallas import tpu as pltpu
from jax.experimental.pallas import tpu_sc as plsc

_NW = 32
_LANES = 16
_CH = 4


def _sc_gather_sum(ntab, etab, nidx2d, eidx2d, deg, n_pad, d):
    ipc = _CH * deg
    nchunks = n_pad // _CH
    per_w = nchunks // _NW
    nouter = per_w // 2

    mesh = plsc.VectorSubcoreMesh(core_axis_name="c", subcore_axis_name="s")

    @functools.partial(
        pl.kernel,
        mesh=mesh,
        out_type=[jax.ShapeDtypeStruct((n_pad, d), jnp.float32),
                  jax.ShapeDtypeStruct((n_pad, d), jnp.float32)],
        scratch_types=[
            pltpu.VMEM((ipc,), jnp.int32),
            pltpu.VMEM((ipc,), jnp.int32),
            pltpu.VMEM((ipc,), jnp.int32),
            pltpu.VMEM((ipc,), jnp.int32),
            pltpu.VMEM((ipc, d), jnp.float32),
            pltpu.VMEM((ipc, d), jnp.float32),
            pltpu.VMEM((ipc, d), jnp.float32),
            pltpu.VMEM((ipc, d), jnp.float32),
            pltpu.VMEM((_CH, d), jnp.float32),
            pltpu.VMEM((_CH, d), jnp.float32),
            pltpu.SemaphoreType.DMA,
            pltpu.SemaphoreType.DMA,
            pltpu.SemaphoreType.DMA,
            pltpu.SemaphoreType.DMA,
        ],
    )
    def sc_kernel(node_hbm, edge_hbm, nidx_hbm, eidx_hbm, gn_hbm, ge_hbm,
                  nidx0, nidx1, eidx0, eidx1, nrows0, nrows1, erows0, erows1,
                  outn_v, oute_v, semn0, semn1, seme0, seme1):
        nidx = (nidx0, nidx1)
        eidx = (eidx0, eidx1)
        nrows = (nrows0, nrows1)
        erows = (erows0, erows1)
        semn = (semn0, semn1)
        seme = (seme0, seme1)
        wid = lax.axis_index("s") * 2 + lax.axis_index("c")
        start = wid * per_w
        start_row = start * _CH

        def load_idx(k, buf):
            pltpu.sync_copy(nidx_hbm.at[start + k], nidx[buf])
            pltpu.sync_copy(eidx_hbm.at[start + k], eidx[buf])

        def fire(buf):
            pltpu.async_copy(node_hbm.at[nidx[buf]], nrows[buf], semn[buf])
            pltpu.async_copy(edge_hbm.at[eidx[buf]], erows[buf], seme[buf])

        def drain(buf):
            pltpu.make_async_copy(node_hbm.at[nidx[buf]], nrows[buf],
                                  semn[buf]).wait()
            pltpu.make_async_copy(edge_hbm.at[eidx[buf]], erows[buf],
                                  seme[buf]).wait()

        def compute(b, k):
            def node_body(node, carry):
                nvec = d // _LANES
                def row_body(r, accs):
                    row = node * deg + r
                    upd_n = tuple(
                        accs[j] + nrows[b][row, pl.ds(_LANES * j, _LANES)]
                        for j in range(nvec))
                    upd_e = tuple(
                        accs[nvec + j] + erows[b][row, pl.ds(_LANES * j, _LANES)]
                        for j in range(nvec))
                    return upd_n + upd_e
                zeros = tuple(jnp.zeros((_LANES,), jnp.float32)
                              for _ in range(2 * nvec))
                accs = lax.fori_loop(0, deg, row_body, zeros)
                for j in range(nvec):
                    outn_v[node, pl.ds(_LANES * j, _LANES)] = accs[j]
                    oute_v[node, pl.ds(_LANES * j, _LANES)] = accs[nvec + j]
                return carry
            lax.fori_loop(0, _CH, node_body, 0)
            pltpu.sync_copy(outn_v, gn_hbm.at[pl.ds(start_row + k * _CH, _CH), :])
            pltpu.sync_copy(oute_v, ge_hbm.at[pl.ds(start_row + k * _CH, _CH), :])

        load_idx(0, 0)
        fire(0)

        def outer(i, carry):
            for b in range(2):
                k = 2 * i + b
                if b == 0:
                    load_idx(k + 1, 1)
                    fire(1)
                else:
                    @pl.when(i < nouter - 1)
                    def _():
                        load_idx(k + 1, 0)
                        fire(0)
                drain(b)
                compute(b, k)
            return carry

        lax.fori_loop(0, nouter, outer, 0)

    return sc_kernel(ntab, etab, nidx2d, eidx2d)


def _tc_finish(node_feats, gn, ge, w, b2, scale):
    n, d = node_feats.shape
    h = w.shape[0]
    br = 2000
    grid = (n // br,)

    def body(x_ref, gn_ref, ge_ref, w_ref, b_ref, o_ref):
        xs = x_ref[...] + gn_ref[...]
        acc = lax.dot_general(xs, w_ref[...], (((1,), (1,)), ((), ())),
                              preferred_element_type=jnp.float32)
        o_ref[...] = acc + ge_ref[...] + scale * b_ref[...]

    return pl.pallas_call(
        body,
        grid=grid,
        in_specs=[
            pl.BlockSpec((br, d), lambda i: (i, 0)),
            pl.BlockSpec((br, h), lambda i: (i, 0)),
            pl.BlockSpec((br, h), lambda i: (i, 0)),
            pl.BlockSpec((h, d), lambda i: (0, 0)),
            pl.BlockSpec((1, h), lambda i: (0, 0)),
        ],
        out_specs=pl.BlockSpec((br, h), lambda i: (i, 0)),
        out_shape=jax.ShapeDtypeStruct((n, h), jnp.float32),
    )(node_feats, gn, ge, w, b2)


def kernel(node_feats, edge_feats, neighbors, W, b):
    n, d = node_feats.shape
    deg = neighbors.shape[1]
    ipc = _CH * deg
    chunks_per_w = -(-(n // _CH) // (2 * _NW)) * 2
    n_pad = chunks_per_w * _NW * _CH
    nchunks = n_pad // _CH
    nidx = neighbors[:, :, 0].reshape(n * deg)
    eidx = neighbors[:, :, 1].reshape(n * deg)
    pad = nchunks * ipc - n * deg
    tail = jnp.arange(pad, dtype=jnp.int32) % n
    nidx2d = jnp.concatenate([nidx, tail]).reshape(nchunks, ipc)
    eidx2d = jnp.concatenate([eidx, tail]).reshape(nchunks, ipc)
    gn, ge = _sc_gather_sum(node_feats, edge_feats, nidx2d, eidx2d,
                            deg, n_pad, d)
    return _tc_finish(node_feats, gn, ge, W, b.reshape(1, -1), float(deg + 1))
